# baseline (speedup 1.0000x reference)
import functools

import jax
import jax.numpy as jnp
from jax import lax
from jax.experimental import pallas as pl
from jax.experimental.pallas import tpu as pltpu
from jax.experimental.pallas import tpu_sc as plsc

CHUNK = 128
NBUF = 4


@functools.lru_cache(maxsize=None)
def _build(total, dim):
    info = plsc.get_sparse_core_info()
    nw = info.num_cores * info.num_subcores
    nc = info.num_cores

    half = CHUNK // 2
    wide = CHUNK * dim // 128
    n_chunks = total // CHUNK
    chunks_per_w = n_chunks // nw
    n_outer = chunks_per_w // NBUF
    rem = chunks_per_w - n_outer * NBUF

    mesh = plsc.VectorSubcoreMesh(core_axis_name="c", subcore_axis_name="s")

    @functools.partial(
        pl.kernel,
        mesh=mesh,
        compiler_params=pltpu.CompilerParams(use_tc_tiling_on_sc=False),
        out_type=jax.ShapeDtypeStruct((total * dim // 128, 128), jnp.float32),
        scratch_types=[
            pltpu.VMEM((chunks_per_w, 2, half), jnp.int32),
            pltpu.VMEM((NBUF, 2, half, dim), jnp.float32),
        ]
        + [pltpu.SemaphoreType.DMA] * (2 * NBUF),
    )
    def gather_kernel(x_hbm, table_hbm, out_hbm, idx_v, rows_v, *sems):
        gsems, osems = sems[:NBUF], sems[NBUF:]
        wid = lax.axis_index("s") * nc + lax.axis_index("c")
        base_wide = wid * chunks_per_w * wide

        def out_slot(c):
            return out_hbm.at[pl.ds(base_wide + c * wide, wide)]

        def fire_gather(c, b):
            for h in range(2):
                pltpu.async_copy(
                    table_hbm.at[idx_v.at[c, h]],
                    rows_v.at[b, h],
                    gsems[b],
                )

        def wait_gather(c, b):
            for h in range(2):
                pltpu.make_async_copy(
                    table_hbm.at[idx_v.at[c, h]],
                    rows_v.at[b, h],
                    gsems[b],
                ).wait()

        def fire_wb(c, b):
            for h in range(2):
                pltpu.async_copy(
                    rows_v.at[b, h],
                    out_slot(c).at[:, pl.ds(h * dim, dim)],
                    osems[b],
                )

        def wait_wb(c, b):
            for h in range(2):
                pltpu.make_async_copy(
                    rows_v.at[b, h],
                    out_slot(c).at[:, pl.ds(h * dim, dim)],
                    osems[b],
                ).wait()

        pltpu.sync_copy(x_hbm.at[wid], idx_v)

        for b in range(NBUF):
            fire_gather(b, b)

        def outer(g, carry):
            for b in range(NBUF):
                c = g * NBUF + b
                wait_gather(c, b)
                fire_wb(c, b)
                nxt = c + NBUF

                @pl.when(nxt < chunks_per_w)
                def _():
                    wait_wb(c, b)
                    fire_gather(nxt, b)

            return carry

        lax.fori_loop(0, n_outer, outer, 0)

        for b in range(rem):
            c = n_outer * NBUF + b
            wait_gather(c, b)
            fire_wb(c, b)

        for b in range(NBUF):
            c = chunks_per_w - NBUF + b
            wait_wb(c, b)

    return gather_kernel


def kernel(x, table):
    batch, fields = x.shape
    total = batch * fields
    dim = table.shape[1]
    nw = 32
    chunks_per_w = total // (nw * CHUNK)
    xf = (
        x.reshape(nw, chunks_per_w, CHUNK // 2, 2)
        .transpose(0, 1, 3, 2)
        .reshape(nw, chunks_per_w, 2, CHUNK // 2)
    )
    out = _build(total, dim)(xf, table)
    return out.reshape(batch, fields, dim)


# speedup vs baseline: 1.1307x; 1.1307x over previous
import functools

import jax
import jax.numpy as jnp
from jax import lax
from jax.experimental import pallas as pl
from jax.experimental.pallas import tpu as pltpu
from jax.experimental.pallas import tpu_sc as plsc

BPC = 4
NBUF = 4


@functools.lru_cache(maxsize=None)
def _build(batch, fields, dim):
    info = plsc.get_sparse_core_info()
    nw = info.num_cores * info.num_subcores
    nc = info.num_cores

    rows_per_chunk = BPC * fields
    batches_per_w = batch // nw
    chunks_per_w = batches_per_w // BPC
    n_outer = chunks_per_w // NBUF
    rem = chunks_per_w - n_outer * NBUF

    mesh = plsc.VectorSubcoreMesh(core_axis_name="c", subcore_axis_name="s")

    @functools.partial(
        pl.kernel,
        mesh=mesh,
        compiler_params=pltpu.CompilerParams(use_tc_tiling_on_sc=False),
        out_type=jax.ShapeDtypeStruct((batch, fields, dim), jnp.float32),
        scratch_types=[
            pltpu.VMEM((chunks_per_w, rows_per_chunk), jnp.int32),
            pltpu.VMEM((NBUF, rows_per_chunk, dim), jnp.float32),
        ]
        + [pltpu.SemaphoreType.DMA] * (2 * NBUF),
    )
    def gather_kernel(x_hbm, table_hbm, out_hbm, idx_v, rows_v, *sems):
        gsems, osems = sems[:NBUF], sems[NBUF:]
        wid = lax.axis_index("s") * nc + lax.axis_index("c")
        base_batch = wid * batches_per_w

        def fire_gather(c, b):
            pltpu.async_copy(
                table_hbm.at[idx_v.at[c]], rows_v.at[b], gsems[b]
            )

        def wait_gather(c, b):
            pltpu.make_async_copy(
                table_hbm.at[idx_v.at[c]], rows_v.at[b], gsems[b]
            ).wait()

        def fire_wb(c, b):
            for k in range(BPC):
                pltpu.async_copy(
                    rows_v.at[b, pl.ds(k * fields, fields)],
                    out_hbm.at[base_batch + c * BPC + k],
                    osems[b],
                )

        def wait_wb(c, b):
            for k in range(BPC):
                pltpu.make_async_copy(
                    rows_v.at[b, pl.ds(k * fields, fields)],
                    out_hbm.at[base_batch + c * BPC + k],
                    osems[b],
                ).wait()

        pltpu.sync_copy(x_hbm.at[wid], idx_v)

        for b in range(NBUF):
            fire_gather(b, b)

        def outer(g, carry):
            for b in range(NBUF):
                c = g * NBUF + b
                wait_gather(c, b)
                fire_wb(c, b)
                nxt = c + NBUF

                @pl.when(nxt < chunks_per_w)
                def _():
                    wait_wb(c, b)
                    fire_gather(nxt, b)

            return carry

        lax.fori_loop(0, n_outer, outer, 0)

        for b in range(rem):
            c = n_outer * NBUF + b
            wait_gather(c, b)
            fire_wb(c, b)

        for b in range(NBUF):
            c = chunks_per_w - NBUF + b
            wait_wb(c, b)

    return gather_kernel


def kernel(x, table):
    batch, fields = x.shape
    dim = table.shape[1]
    nw = 32
    chunks_per_w = batch // (nw * BPC)
    xf = x.reshape(nw, chunks_per_w, BPC * fields)
    return _build(batch, fields, dim)(xf, table)


# speedup vs baseline: 1.5269x; 1.3504x over previous
import functools

import jax
import jax.numpy as jnp
from jax import lax
from jax.experimental import pallas as pl
from jax.experimental.pallas import tpu as pltpu
from jax.experimental.pallas import tpu_sc as plsc

BPC = 4
NBUF = 4


@functools.lru_cache(maxsize=None)
def _build(batch, fields, dim):
    info = plsc.get_sparse_core_info()
    nw = info.num_cores * info.num_subcores
    nc = info.num_cores

    rows_per_chunk = BPC * fields
    batches_per_w = batch // nw
    chunks_per_w = batches_per_w // BPC
    n_outer = chunks_per_w // NBUF
    rem = chunks_per_w - n_outer * NBUF

    mesh = plsc.VectorSubcoreMesh(core_axis_name="c", subcore_axis_name="s")

    @functools.partial(
        pl.kernel,
        mesh=mesh,
        compiler_params=pltpu.CompilerParams(use_tc_tiling_on_sc=False),
        out_type=jax.ShapeDtypeStruct((batch, 32, 128), jnp.float32),
        scratch_types=[
            pltpu.VMEM((chunks_per_w, rows_per_chunk), jnp.int32),
            pltpu.VMEM((NBUF, rows_per_chunk, dim), jnp.float32),
        ]
        + [pltpu.SemaphoreType.DMA] * (2 * NBUF),
    )
    def gather_kernel(x_hbm, table_hbm, out_hbm, idx_v, rows_v, *sems):
        gsems, osems = sems[:NBUF], sems[NBUF:]
        wid = lax.axis_index("s") * nc + lax.axis_index("c")
        base_batch = wid * batches_per_w

        def fire_gather(c, b):
            pltpu.async_copy(
                table_hbm.at[idx_v.at[c]], rows_v.at[b], gsems[b]
            )

        def wait_gather(c, b):
            pltpu.make_async_copy(
                table_hbm.at[idx_v.at[c]], rows_v.at[b], gsems[b]
            ).wait()

        def fire_wb(c, b):
            for k in range(BPC):
                pltpu.async_copy(
                    rows_v.at[b, pl.ds(k * fields, fields)],
                    out_hbm.at[base_batch + c * BPC + k, pl.ds(0, fields), pl.ds(0, dim)],
                    osems[b],
                )

        def wait_wb(c, b):
            for k in range(BPC):
                pltpu.make_async_copy(
                    rows_v.at[b, pl.ds(k * fields, fields)],
                    out_hbm.at[base_batch + c * BPC + k, pl.ds(0, fields), pl.ds(0, dim)],
                    osems[b],
                ).wait()

        pltpu.sync_copy(x_hbm.at[wid], idx_v)

        for b in range(NBUF):
            fire_gather(b, b)

        def outer(g, carry):
            for b in range(NBUF):
                c = g * NBUF + b
                wait_gather(c, b)
                fire_wb(c, b)
                nxt = c + NBUF

                @pl.when(nxt < chunks_per_w)
                def _():
                    wait_wb(c, b)
                    fire_gather(nxt, b)

            return carry

        lax.fori_loop(0, n_outer, outer, 0)

        for b in range(rem):
            c = n_outer * NBUF + b
            wait_gather(c, b)
            fire_wb(c, b)

        for b in range(NBUF):
            c = chunks_per_w - NBUF + b
            wait_wb(c, b)

    return gather_kernel


def kernel(x, table):
    batch, fields = x.shape
    dim = table.shape[1]
    nw = 32
    chunks_per_w = batch // (nw * BPC)
    xf = x.reshape(nw, chunks_per_w, BPC * fields)
    out = _build(batch, fields, dim)(xf, table)
    return out[:, :fields, :dim]


# speedup vs baseline: 1.5328x; 1.0038x over previous
import functools

import jax
import jax.numpy as jnp
from jax import lax
from jax.experimental import pallas as pl
from jax.experimental.pallas import tpu as pltpu
from jax.experimental.pallas import tpu_sc as plsc

BPC = 4
NBUF = 4


@functools.lru_cache(maxsize=None)
def _build(batch, fields, dim):
    info = plsc.get_sparse_core_info()
    nw = info.num_cores * info.num_subcores
    nc = info.num_cores

    rows_per_chunk = BPC * fields
    batches_per_w = batch // nw
    chunks_per_w = batches_per_w // BPC
    n_outer = chunks_per_w // NBUF
    rem = chunks_per_w - n_outer * NBUF

    mesh = plsc.VectorSubcoreMesh(core_axis_name="c", subcore_axis_name="s")

    @functools.partial(
        pl.kernel,
        mesh=mesh,
        compiler_params=pltpu.CompilerParams(use_tc_tiling_on_sc=False),
        out_type=jax.ShapeDtypeStruct((batch, 32, 128), jnp.float32),
        scratch_types=[
            pltpu.VMEM((chunks_per_w, rows_per_chunk), jnp.int32),
            pltpu.VMEM((NBUF, rows_per_chunk, dim), jnp.float32),
        ]
        + [pltpu.SemaphoreType.DMA] * (2 * NBUF),
    )
    def gather_kernel(x_hbm, table_hbm, out_hbm, idx_v, rows_v, *sems):
        gsems, osems = sems[:NBUF], sems[NBUF:]
        wid = lax.axis_index("s") * nc + lax.axis_index("c")
        base_batch = wid * batches_per_w

        def fire_gather(c, b):
            pltpu.async_copy(
                table_hbm.at[idx_v.at[c]], rows_v.at[b], gsems[b]
            )

        def wait_gather(c, b):
            pltpu.make_async_copy(
                table_hbm.at[idx_v.at[c]], rows_v.at[b], gsems[b]
            ).wait()

        def fire_wb(c, b):
            for k in range(BPC):
                pltpu.async_copy(
                    rows_v.at[b, pl.ds(k * fields, fields)],
                    out_hbm.at[base_batch + c * BPC + k, pl.ds(0, fields), pl.ds(0, dim)],
                    osems[b],
                )

        def wait_wb(c, b):
            for k in range(BPC):
                pltpu.make_async_copy(
                    rows_v.at[b, pl.ds(k * fields, fields)],
                    out_hbm.at[base_batch + c * BPC + k, pl.ds(0, fields), pl.ds(0, dim)],
                    osems[b],
                ).wait()

        pltpu.sync_copy(x_hbm.at[wid], idx_v)

        for b in range(NBUF):
            fire_gather(b, b)

        def outer(g, carry):
            for b in range(NBUF):
                c = g * NBUF + b
                wait_gather(c, b)
                fire_wb(c, b)
                nxt = c + NBUF

                @pl.when(nxt < chunks_per_w)
                def _():
                    wait_wb(c, b)
                    fire_gather(nxt, b)

            return carry

        lax.fori_loop(0, n_outer, outer, 0)

        for b in range(rem):
            c = n_outer * NBUF + b
            wait_gather(c, b)
            fire_wb(c, b)

        for b in range(NBUF):
            c = chunks_per_w - NBUF + b
            wait_wb(c, b)

    return gather_kernel


def kernel(x, table):
    batch, fields = x.shape
    dim = table.shape[1]
    nw = 32
    chunks_per_w = batch // (nw * BPC)
    xf = jnp.maximum(x, 0).reshape(nw, chunks_per_w, BPC * fields)
    out = _build(batch, fields, dim)(xf, table)
    return out[:, :fields, :dim]
